# baseline (speedup 1.0000x reference)
import functools

import jax
import jax.numpy as jnp
from jax import lax
from jax.experimental import pallas as pl
from jax.experimental.pallas import tpu as pltpu
from jax.experimental.pallas import tpu_sc as plsc

N_NODES = 10000
N_EDGES = 320000
D = 128

NC = 2
NS = 16
NW = NC * NS
N_PAD = 10240
EPT = N_EDGES // NW
CHUNK = 125
NCHUNK = EPT // CHUNK
NBLK = 2
BCHUNK = NCHUNK // NBLK
ROWS_PT = N_PAD // NS
ROW_BLK = 1024
N_ROW_BLKS = N_PAD // ROW_BLK

_mesh = plsc.VectorSubcoreMesh(core_axis_name="c", subcore_axis_name="s")


@functools.partial(
    pl.kernel,
    out_type=jax.ShapeDtypeStruct((NC, N_PAD), jnp.float32),
    mesh=_mesh,
    scratch_types=[
        pltpu.VMEM((BCHUNK, CHUNK), jnp.int32),
        pltpu.VMEM((CHUNK,), jnp.float32),
        pltpu.VMEM_SHARED((N_PAD,), jnp.float32),
    ],
)
def _deg_kernel(ei_hbm, ones_hbm, zeros1_hbm, out_hbm, dstv, onesv, acc1):
    cid = lax.axis_index("c")
    sid = lax.axis_index("s")
    wid = sid * NC + cid
    pltpu.sync_copy(zeros1_hbm.at[pl.ds(sid * ROWS_PT, ROWS_PT)],
                    acc1.at[pl.ds(sid * ROWS_PT, ROWS_PT)])
    pltpu.sync_copy(ones_hbm, onesv)
    plsc.subcore_barrier()

    for b in range(NBLK):
        pltpu.sync_copy(ei_hbm.at[1, wid, b, :, :], dstv)

        def body(c, carry):
            pltpu.sync_copy(onesv, acc1.at[dstv.at[c]], add=True)
            return carry

        lax.fori_loop(0, BCHUNK, body, 0)
    plsc.subcore_barrier()
    pltpu.sync_copy(acc1.at[pl.ds(sid * ROWS_PT, ROWS_PT)],
                    out_hbm.at[cid, pl.ds(sid * ROWS_PT, ROWS_PT)])


@functools.partial(
    pl.kernel,
    out_type=jax.ShapeDtypeStruct((NC, N_PAD, D), jnp.float32),
    mesh=_mesh,
    scratch_types=[
        pltpu.VMEM((BCHUNK, CHUNK), jnp.int32),
        pltpu.VMEM((BCHUNK, CHUNK), jnp.int32),
        pltpu.VMEM((CHUNK, D), jnp.float32),
        pltpu.VMEM((CHUNK, D), jnp.float32),
        pltpu.VMEM_SHARED((N_PAD, D), jnp.float32),
        pltpu.SemaphoreType.DMA,
        pltpu.SemaphoreType.DMA,
        pltpu.SemaphoreType.DMA,
        pltpu.SemaphoreType.DMA,
    ],
)
def _edge_scatter_kernel(hp_hbm, ei_hbm, zeros2_hbm, out_hbm,
                         srcv, dstv, bufa, bufb, acc, sema, semb, semc, semd):
    cid = lax.axis_index("c")
    sid = lax.axis_index("s")
    wid = sid * NC + cid
    for b in range(NBLK):
        pltpu.sync_copy(ei_hbm.at[0, wid, b, :, :], srcv)
        pltpu.sync_copy(ei_hbm.at[1, wid, b, :, :], dstv)
        pltpu.async_copy(hp_hbm.at[srcv.at[0]], bufa, sema)
        pltpu.async_copy(hp_hbm.at[srcv.at[1]], bufb, semb)
        if b == 0:
            pltpu.sync_copy(zeros2_hbm.at[pl.ds(sid * ROWS_PT, ROWS_PT)],
                            acc.at[pl.ds(sid * ROWS_PT, ROWS_PT)])
            plsc.subcore_barrier()

        def body(g, carry):
            c0 = 2 * g
            c1 = c0 + 1
            pltpu.make_async_copy(hp_hbm.at[srcv.at[c0]], bufa, sema).wait()
            pltpu.async_copy(bufa, acc.at[dstv.at[c0]], semc, add=True)
            pltpu.make_async_copy(hp_hbm.at[srcv.at[c1]], bufb, semb).wait()
            pltpu.async_copy(bufb, acc.at[dstv.at[c1]], semd, add=True)
            pltpu.make_async_copy(bufa, acc.at[dstv.at[c0]], semc).wait()
            pltpu.async_copy(hp_hbm.at[srcv.at[jnp.minimum(c0 + 2, BCHUNK - 1)]],
                             bufa, sema)
            pltpu.make_async_copy(bufb, acc.at[dstv.at[c1]], semd).wait()
            pltpu.async_copy(hp_hbm.at[srcv.at[jnp.minimum(c1 + 2, BCHUNK - 1)]],
                             bufb, semb)
            return carry

        lax.fori_loop(0, BCHUNK // 2, body, 0)
        pltpu.make_async_copy(hp_hbm.at[srcv.at[0]], bufa, sema).wait()
        pltpu.make_async_copy(hp_hbm.at[srcv.at[0]], bufb, semb).wait()
    plsc.subcore_barrier()
    pltpu.sync_copy(acc.at[pl.ds(sid * ROWS_PT, ROWS_PT)],
                    out_hbm.at[cid, pl.ds(sid * ROWS_PT, ROWS_PT)])


def _tc1_body(deg_ref, x_ref, w_ref, dis_ref, hp_ref):
    deg = deg_ref[0, :] + deg_ref[1, :] + 1.0
    dis = lax.rsqrt(deg)
    dis_ref[:] = dis
    hp_ref[:, :] = jnp.dot(x_ref[:, :], w_ref[:, :],
                           preferred_element_type=jnp.float32) * dis[:, None]


def _tc2_body(p_ref, hp1_ref, dis_ref, b1_ref, w2_ref, hp2_ref):
    dis = dis_ref[:]
    agg = p_ref[0, :, :] + p_ref[1, :, :] + hp1_ref[:, :]
    z = jnp.maximum(agg * dis[:, None] + b1_ref[:][None, :], 0.0)
    hp2_ref[:, :] = jnp.dot(z, w2_ref[:, :],
                            preferred_element_type=jnp.float32) * dis[:, None]


def _tc3_body(p_ref, hp2_ref, dis_ref, b2_ref, out_ref):
    dis = dis_ref[:]
    agg = p_ref[0, :, :] + p_ref[1, :, :] + hp2_ref[:, :]
    out_ref[:, :] = agg * dis[:, None] + b2_ref[:][None, :]


_blk_rows2 = pl.BlockSpec((ROW_BLK, D), lambda i: (i, 0))
_blk_part = pl.BlockSpec((NC, ROW_BLK, D), lambda i: (0, i, 0))
_blk_dis = pl.BlockSpec((ROW_BLK,), lambda i: (i,))
_blk_w = pl.BlockSpec((D, D), lambda i: (0, 0))
_blk_b = pl.BlockSpec((D,), lambda i: (0,))

_tc1 = pl.pallas_call(
    _tc1_body,
    grid=(N_ROW_BLKS,),
    in_specs=[pl.BlockSpec((NC, ROW_BLK), lambda i: (0, i)), _blk_rows2, _blk_w],
    out_specs=[_blk_dis, _blk_rows2],
    out_shape=[jax.ShapeDtypeStruct((N_PAD,), jnp.float32),
               jax.ShapeDtypeStruct((N_PAD, D), jnp.float32)],
)

_tc2 = pl.pallas_call(
    _tc2_body,
    grid=(N_ROW_BLKS,),
    in_specs=[_blk_part, _blk_rows2, _blk_dis, _blk_b, _blk_w],
    out_specs=_blk_rows2,
    out_shape=jax.ShapeDtypeStruct((N_PAD, D), jnp.float32),
)

_tc3 = pl.pallas_call(
    _tc3_body,
    grid=(N_ROW_BLKS,),
    in_specs=[_blk_part, _blk_rows2, _blk_dis, _blk_b],
    out_specs=_blk_rows2,
    out_shape=jax.ShapeDtypeStruct((N_NODES, D), jnp.float32),
)


def kernel(x, edge_index, W1, b1, W2, b2):
    ei = edge_index.astype(jnp.int32).reshape(2, NW, NBLK, BCHUNK, CHUNK)

    x_pad = jnp.zeros((N_PAD, D), jnp.float32).at[:N_NODES, :].set(x)
    zeros1 = jnp.zeros((N_PAD,), jnp.float32)
    zeros2 = jnp.zeros((N_PAD, D), jnp.float32)
    ones = jnp.ones((CHUNK,), jnp.float32)

    deg_p = _deg_kernel(ei, ones, zeros1)
    dis, hp1 = _tc1(deg_p, x_pad, W1)
    p1 = _edge_scatter_kernel(hp1, ei, zeros2)
    hp2 = _tc2(p1, hp1, dis, b1, W2)
    p2 = _edge_scatter_kernel(hp2, ei, zeros2)
    return _tc3(p2, hp2, dis, b2)


# speedup vs baseline: 1.2768x; 1.2768x over previous
import functools

import jax
import jax.numpy as jnp
from jax import lax
from jax.experimental import pallas as pl
from jax.experimental.pallas import tpu as pltpu
from jax.experimental.pallas import tpu_sc as plsc

N_NODES = 10000
N_EDGES = 320000
D = 128

NC = 2
NS = 16
NW = NC * NS
N_PAD = 10240
EPT = N_EDGES // NW
CHUNK = 125
NCHUNK = EPT // CHUNK
NBLK = 2
BCHUNK = NCHUNK // NBLK
ROWS_PT = N_PAD // NS
ROW_BLK = 1024
N_ROW_BLKS = N_PAD // ROW_BLK

_mesh = plsc.VectorSubcoreMesh(core_axis_name="c", subcore_axis_name="s")


@functools.partial(
    pl.kernel,
    out_type=jax.ShapeDtypeStruct((NC, N_PAD), jnp.float32),
    mesh=_mesh,
    scratch_types=[
        pltpu.VMEM((NBLK, BCHUNK, CHUNK), jnp.int32),
        pltpu.VMEM((CHUNK,), jnp.float32),
        pltpu.VMEM_SHARED((N_PAD,), jnp.float32),
        pltpu.SemaphoreType.DMA,
    ],
)
def _deg_kernel(ei_hbm, ones_hbm, zeros1_hbm, out_hbm, dstv, onesv, acc1, sem):
    cid = lax.axis_index("c")
    sid = lax.axis_index("s")
    wid = sid * NC + cid
    pltpu.sync_copy(zeros1_hbm.at[pl.ds(sid * ROWS_PT, ROWS_PT)],
                    acc1.at[pl.ds(sid * ROWS_PT, ROWS_PT)])
    pltpu.sync_copy(ei_hbm.at[0 + 1, wid, :, :, :], dstv)
    pltpu.sync_copy(ones_hbm, onesv)
    plsc.subcore_barrier()

    for b in range(NBLK):

        def issue(c, carry):
            pltpu.async_copy(onesv, acc1.at[dstv.at[b, c]], sem, add=True)
            return carry

        lax.fori_loop(0, BCHUNK, issue, 0)
    for b in range(NBLK):

        def drain(c, carry):
            pltpu.make_async_copy(onesv, acc1.at[dstv.at[b, c]], sem).wait()
            return carry

        lax.fori_loop(0, BCHUNK, drain, 0)
    plsc.subcore_barrier()
    pltpu.sync_copy(acc1.at[pl.ds(sid * ROWS_PT, ROWS_PT)],
                    out_hbm.at[cid, pl.ds(sid * ROWS_PT, ROWS_PT)])


@functools.partial(
    pl.kernel,
    out_type=jax.ShapeDtypeStruct((NC, N_PAD, D), jnp.float32),
    mesh=_mesh,
    scratch_types=[
        pltpu.VMEM((BCHUNK, CHUNK), jnp.int32),
        pltpu.VMEM((BCHUNK, CHUNK), jnp.int32),
        pltpu.VMEM((CHUNK, D), jnp.float32),
        pltpu.VMEM((CHUNK, D), jnp.float32),
        pltpu.VMEM_SHARED((N_PAD, D), jnp.float32),
        pltpu.SemaphoreType.DMA,
        pltpu.SemaphoreType.DMA,
    ],
)
def _edge_scatter_kernel(hp_hbm, ei_hbm, zeros2_hbm, out_hbm,
                         srcv, dstv, bufa, bufb, acc, sema, semb):
    cid = lax.axis_index("c")
    sid = lax.axis_index("s")
    wid = sid * NC + cid
    for b in range(NBLK):
        pltpu.sync_copy(ei_hbm.at[0, wid, b, :, :], srcv)
        pltpu.sync_copy(ei_hbm.at[1, wid, b, :, :], dstv)
        pltpu.async_copy(hp_hbm.at[srcv.at[0]], bufa, sema)
        pltpu.async_copy(hp_hbm.at[srcv.at[1]], bufb, semb)
        if b == 0:
            pltpu.sync_copy(zeros2_hbm.at[pl.ds(sid * ROWS_PT, ROWS_PT)],
                            acc.at[pl.ds(sid * ROWS_PT, ROWS_PT)])
            plsc.subcore_barrier()

        def body(g, carry):
            c0 = 2 * g
            c1 = c0 + 1
            pltpu.make_async_copy(hp_hbm.at[srcv.at[c0]], bufa, sema).wait()
            pltpu.sync_copy(bufa, acc.at[dstv.at[c0]], add=True)
            pltpu.async_copy(hp_hbm.at[srcv.at[jnp.minimum(c0 + 2, BCHUNK - 1)]],
                             bufa, sema)
            pltpu.make_async_copy(hp_hbm.at[srcv.at[c1]], bufb, semb).wait()
            pltpu.sync_copy(bufb, acc.at[dstv.at[c1]], add=True)
            pltpu.async_copy(hp_hbm.at[srcv.at[jnp.minimum(c1 + 2, BCHUNK - 1)]],
                             bufb, semb)
            return carry

        lax.fori_loop(0, BCHUNK // 2, body, 0)
        pltpu.make_async_copy(hp_hbm.at[srcv.at[0]], bufa, sema).wait()
        pltpu.make_async_copy(hp_hbm.at[srcv.at[0]], bufb, semb).wait()
    plsc.subcore_barrier()
    pltpu.sync_copy(acc.at[pl.ds(sid * ROWS_PT, ROWS_PT)],
                    out_hbm.at[cid, pl.ds(sid * ROWS_PT, ROWS_PT)])


def _tc1_body(deg_ref, x_ref, w_ref, dis_ref, hp_ref):
    deg = deg_ref[0, :] + deg_ref[1, :] + 1.0
    dis = lax.rsqrt(deg)
    dis_ref[:] = dis
    hp_ref[:, :] = jnp.dot(x_ref[:, :], w_ref[:, :],
                           preferred_element_type=jnp.float32) * dis[:, None]


def _tc2_body(p_ref, hp1_ref, dis_ref, b1_ref, w2_ref, hp2_ref):
    dis = dis_ref[:]
    agg = p_ref[0, :, :] + p_ref[1, :, :] + hp1_ref[:, :]
    z = jnp.maximum(agg * dis[:, None] + b1_ref[:][None, :], 0.0)
    hp2_ref[:, :] = jnp.dot(z, w2_ref[:, :],
                            preferred_element_type=jnp.float32) * dis[:, None]


def _tc3_body(p_ref, hp2_ref, dis_ref, b2_ref, out_ref):
    dis = dis_ref[:]
    agg = p_ref[0, :, :] + p_ref[1, :, :] + hp2_ref[:, :]
    out_ref[:, :] = agg * dis[:, None] + b2_ref[:][None, :]


_blk_rows2 = pl.BlockSpec((ROW_BLK, D), lambda i: (i, 0))
_blk_part = pl.BlockSpec((NC, ROW_BLK, D), lambda i: (0, i, 0))
_blk_dis = pl.BlockSpec((ROW_BLK,), lambda i: (i,))
_blk_w = pl.BlockSpec((D, D), lambda i: (0, 0))
_blk_b = pl.BlockSpec((D,), lambda i: (0,))

_tc1 = pl.pallas_call(
    _tc1_body,
    grid=(N_ROW_BLKS,),
    in_specs=[pl.BlockSpec((NC, ROW_BLK), lambda i: (0, i)), _blk_rows2, _blk_w],
    out_specs=[_blk_dis, _blk_rows2],
    out_shape=[jax.ShapeDtypeStruct((N_PAD,), jnp.float32),
               jax.ShapeDtypeStruct((N_PAD, D), jnp.float32)],
)

_tc2 = pl.pallas_call(
    _tc2_body,
    grid=(N_ROW_BLKS,),
    in_specs=[_blk_part, _blk_rows2, _blk_dis, _blk_b, _blk_w],
    out_specs=_blk_rows2,
    out_shape=jax.ShapeDtypeStruct((N_PAD, D), jnp.float32),
)

_tc3 = pl.pallas_call(
    _tc3_body,
    grid=(N_ROW_BLKS,),
    in_specs=[_blk_part, _blk_rows2, _blk_dis, _blk_b],
    out_specs=_blk_rows2,
    out_shape=jax.ShapeDtypeStruct((N_NODES, D), jnp.float32),
)


def kernel(x, edge_index, W1, b1, W2, b2):
    ei = edge_index.astype(jnp.int32).reshape(2, NW, NBLK, BCHUNK, CHUNK)

    x_pad = jnp.zeros((N_PAD, D), jnp.float32).at[:N_NODES, :].set(x)
    zeros1 = jnp.zeros((N_PAD,), jnp.float32)
    zeros2 = jnp.zeros((N_PAD, D), jnp.float32)
    ones = jnp.ones((CHUNK,), jnp.float32)

    deg_p = _deg_kernel(ei, ones, zeros1)
    dis, hp1 = _tc1(deg_p, x_pad, W1)
    p1 = _edge_scatter_kernel(hp1, ei, zeros2)
    hp2 = _tc2(p1, hp1, dis, b1, W2)
    p2 = _edge_scatter_kernel(hp2, ei, zeros2)
    return _tc3(p2, hp2, dis, b2)
